# baseline (speedup 1.0000x reference)
import jax
import jax.numpy as jnp
from jax import lax
from jax.experimental import pallas as pl

_B = 8
_N = 5000
_C = 80
_D = 100
_NEG = -1e30


def _reduce_body(s_ref, best_ref, cls_ref):
    s = s_ref[0]
    m = jnp.max(s, axis=0, keepdims=True)
    ci = lax.broadcasted_iota(jnp.int32, (_C, _N), 0)
    c = jnp.min(jnp.where(s == m, ci, _C), axis=0, keepdims=True)
    best_ref[0] = m
    cls_ref[0] = c


def _nms_body(best_ref, cls_ref, boxes_ref,
              idx_ref, sc_ref, x1_ref, y1_ref, x2_ref, y2_ref, cls_out_ref,
              cnt_ref):
    clsf = cls_ref[...].astype(jnp.float32)
    x1 = boxes_ref[0]
    y1 = boxes_ref[1]
    x2 = boxes_ref[2]
    y2 = boxes_ref[3]
    areas = (x2 - x1) * (y2 - y1)
    best = best_ref[...]
    active0 = jnp.where(best > 0.0, best, _NEG)
    colf = lax.broadcasted_iota(jnp.int32, (_B, _N), 1).astype(jnp.float32)
    ocol = lax.broadcasted_iota(jnp.int32, (_B, 128), 1)
    onesv = jnp.ones((_N, 1), jnp.float32)

    zf = jnp.zeros((_B, 128), jnp.float32)
    init = (active0,
            jnp.zeros((_B, 1), jnp.int32),
            jnp.full((_B, 128), -1, jnp.int32),
            zf,
            zf, zf, zf, zf,
            jnp.full((_B, 128), -1, jnp.int32))

    def mm(a):
        return lax.dot_general(a, onesv, (((1,), (0,)), ((), ())),
                               precision=lax.Precision.HIGHEST,
                               preferred_element_type=jnp.float32)

    def step(t, carry):
        active, cnt, oidx, osc, ox1, oy1, ox2, oy2, ocls = carry
        m = jnp.max(active, axis=1, keepdims=True)
        valid = m > -1e29
        eq = active == m
        eqf = jnp.where(eq, 1.0, 0.0)
        a_rows = jnp.concatenate(
            [eqf, eqf * colf, eqf * x1, eqf * y1, eqf * x2, eqf * y2,
             eqf * clsf], axis=0)
        r = mm(a_rows)
        cnte = r[0:_B]
        tie_any = jnp.max(cnte) > 1.5

        def slow(_):
            jjm = jnp.min(jnp.where(eq, colf, float(_N)), axis=1,
                          keepdims=True)
            ohf = jnp.where(colf == jjm, 1.0, 0.0)
            r2 = mm(jnp.concatenate(
                [ohf * x1, ohf * y1, ohf * x2, ohf * y2, ohf * clsf],
                axis=0))
            return (jjm, r2[0:_B], r2[_B:2 * _B], r2[2 * _B:3 * _B],
                    r2[3 * _B:4 * _B], r2[4 * _B:5 * _B])

        def fast(_):
            return (r[_B:2 * _B], r[2 * _B:3 * _B], r[3 * _B:4 * _B],
                    r[4 * _B:5 * _B], r[5 * _B:6 * _B], r[6 * _B:7 * _B])

        jjf, sx1, sy1, sx2, sy2, sclf = lax.cond(tie_any, slow, fast, None)

        sar = (sx2 - sx1) * (sy2 - sy1)
        iw = jnp.minimum(sx2, x2) - jnp.maximum(sx1, x1)
        ih = jnp.minimum(sy2, y2) - jnp.maximum(sy1, y1)
        inter = jnp.maximum(iw, 0.0) * jnp.maximum(ih, 0.0)
        union = sar + areas - inter
        supp = inter > 0.5 * union
        active = jnp.where(supp & valid, _NEG, active)
        cnt = cnt + valid.astype(jnp.int32)

        jj = jjf.astype(jnp.int32)
        scl = sclf.astype(jnp.int32)
        slot = ocol == t
        oidx = jnp.where(slot, jnp.where(valid, jj, -1), oidx)
        osc = jnp.where(slot, jnp.where(valid, m, 0.0), osc)
        ox1 = jnp.where(slot, jnp.where(valid, sx1, 0.0), ox1)
        oy1 = jnp.where(slot, jnp.where(valid, sy1, 0.0), oy1)
        ox2 = jnp.where(slot, jnp.where(valid, sx2, 0.0), ox2)
        oy2 = jnp.where(slot, jnp.where(valid, sy2, 0.0), oy2)
        ocls = jnp.where(slot, jnp.where(valid, scl, -1), ocls)
        return (active, cnt, oidx, osc, ox1, oy1, ox2, oy2, ocls)

    (_, cnt, oidx, osc, ox1, oy1, ox2, oy2, ocls) = lax.fori_loop(
        0, _D, step, init)

    idx_ref[...] = oidx[:, :_D]
    sc_ref[...] = osc[:, :_D]
    x1_ref[...] = ox1[:, :_D]
    y1_ref[...] = oy1[:, :_D]
    x2_ref[...] = ox2[:, :_D]
    y2_ref[...] = oy2[:, :_D]
    cls_out_ref[...] = ocls[:, :_D]
    cnt_ref[...] = cnt


@jax.jit
def kernel(scores, boxes):
    scores_t = jnp.swapaxes(scores, 1, 2)
    best, cls = pl.pallas_call(
        _reduce_body,
        grid=(_B,),
        in_specs=[pl.BlockSpec((1, _C, _N), lambda b: (b, 0, 0))],
        out_specs=[pl.BlockSpec((1, 1, _N), lambda b: (b, 0, 0)),
                   pl.BlockSpec((1, 1, _N), lambda b: (b, 0, 0))],
        out_shape=[jax.ShapeDtypeStruct((_B, 1, _N), jnp.float32),
                   jax.ShapeDtypeStruct((_B, 1, _N), jnp.int32)],
    )(scores_t)
    best = best.reshape(_B, _N)
    cls = cls.reshape(_B, _N)
    boxes_t = jnp.transpose(boxes, (2, 0, 1))

    outs = pl.pallas_call(
        _nms_body,
        out_shape=[jax.ShapeDtypeStruct((_B, _D), jnp.int32),
                   jax.ShapeDtypeStruct((_B, _D), jnp.float32),
                   jax.ShapeDtypeStruct((_B, _D), jnp.float32),
                   jax.ShapeDtypeStruct((_B, _D), jnp.float32),
                   jax.ShapeDtypeStruct((_B, _D), jnp.float32),
                   jax.ShapeDtypeStruct((_B, _D), jnp.float32),
                   jax.ShapeDtypeStruct((_B, _D), jnp.int32),
                   jax.ShapeDtypeStruct((_B, 1), jnp.int32)],
    )(best, cls, boxes_t)
    oidx, osc, ox1, oy1, ox2, oy2, ocls, cnt = outs
    boxes_out = jnp.stack([ox1, oy1, ox2, oy2], axis=-1)
    return oidx, osc, boxes_out, ocls, cnt.reshape(_B)


# speedup vs baseline: 2.3721x; 2.3721x over previous
import functools

import jax
import jax.numpy as jnp
from jax import lax
from jax.experimental import pallas as pl
from jax.experimental.pallas import tpu as pltpu

_B = 8
_N = 5000
_C = 80
_D = 100
_NEG = -1e30


def _reduce_body(s_ref, best_ref, cls_ref):
    s = s_ref[0]
    m = jnp.max(s, axis=0, keepdims=True)
    ci = lax.broadcasted_iota(jnp.int32, (_C, _N), 0)
    c = jnp.min(jnp.where(s == m, ci, _C), axis=0, keepdims=True)
    best_ref[0] = m
    cls_ref[0] = c


def _nms_body(best_ref, cls_ref, boxes_ref,
              idx_ref, sc_ref, x1_ref, y1_ref, x2_ref, y2_ref, cls_out_ref,
              cnt_ref):
    best = best_ref[...]
    clsf = cls_ref[...].astype(jnp.float32)
    x1 = boxes_ref[0]
    y1 = boxes_ref[1]
    x2 = boxes_ref[2]
    y2 = boxes_ref[3]
    areas = (x2 - x1) * (y2 - y1)

    active0 = jnp.where(best > 0.0, best, _NEG)
    colf = lax.broadcasted_iota(jnp.int32, (_B, _N), 1).astype(jnp.float32)
    ocol = lax.broadcasted_iota(jnp.int32, (_B, 128), 1)

    zf = jnp.zeros((_B, 128), jnp.float32)
    init = (active0,
            jnp.zeros((_B, 1), jnp.int32),
            jnp.full((_B, 128), -1, jnp.int32),
            zf,
            zf, zf, zf, zf,
            jnp.full((_B, 128), -1, jnp.int32))

    def step(t, carry):
        active, cnt, oidx, osc, ox1, oy1, ox2, oy2, ocls = carry
        m = jnp.max(active, axis=1, keepdims=True)
        valid = m > -1e29
        eq = active == m

        def gf(mask, v):
            return jnp.sum(jnp.where(mask, v, 0.0), axis=1, keepdims=True)

        jjf = jnp.min(jnp.where(eq, colf, float(_N)), axis=1, keepdims=True)
        cnte = gf(eq, 1.0)
        tie_any = jnp.max(cnte) > 1.5

        def slow(_):
            oh = colf == jjf
            return (gf(oh, x1), gf(oh, y1), gf(oh, x2), gf(oh, y2),
                    gf(oh, clsf))

        def fast(_):
            return (gf(eq, x1), gf(eq, y1), gf(eq, x2), gf(eq, y2),
                    gf(eq, clsf))

        sx1, sy1, sx2, sy2, sclf = lax.cond(tie_any, slow, fast, None)
        jj = jjf.astype(jnp.int32)
        scl = sclf.astype(jnp.int32)

        sar = (sx2 - sx1) * (sy2 - sy1)
        iw = jnp.minimum(sx2, x2) - jnp.maximum(sx1, x1)
        ih = jnp.minimum(sy2, y2) - jnp.maximum(sy1, y1)
        inter = jnp.maximum(iw, 0.0) * jnp.maximum(ih, 0.0)
        union = sar + areas - inter
        supp = inter > 0.5 * union
        active = jnp.where(supp & valid, _NEG, active)
        cnt = cnt + valid.astype(jnp.int32)

        slot = ocol == t
        oidx = jnp.where(slot, jnp.where(valid, jj, -1), oidx)
        osc = jnp.where(slot, jnp.where(valid, m, 0.0), osc)
        ox1 = jnp.where(slot, jnp.where(valid, sx1, 0.0), ox1)
        oy1 = jnp.where(slot, jnp.where(valid, sy1, 0.0), oy1)
        ox2 = jnp.where(slot, jnp.where(valid, sx2, 0.0), ox2)
        oy2 = jnp.where(slot, jnp.where(valid, sy2, 0.0), oy2)
        ocls = jnp.where(slot, jnp.where(valid, scl, -1), ocls)
        return (active, cnt, oidx, osc, ox1, oy1, ox2, oy2, ocls)

    (_, cnt, oidx, osc, ox1, oy1, ox2, oy2, ocls) = lax.fori_loop(
        0, _D, step, init)

    idx_ref[...] = oidx[:, :_D]
    sc_ref[...] = osc[:, :_D]
    x1_ref[...] = ox1[:, :_D]
    y1_ref[...] = oy1[:, :_D]
    x2_ref[...] = ox2[:, :_D]
    y2_ref[...] = oy2[:, :_D]
    cls_out_ref[...] = ocls[:, :_D]
    cnt_ref[...] = cnt


@jax.jit
def kernel(scores, boxes):
    scores_t = jnp.swapaxes(scores, 1, 2)
    best, cls = pl.pallas_call(
        _reduce_body,
        grid=(_B,),
        in_specs=[pl.BlockSpec((1, _C, _N), lambda b: (b, 0, 0))],
        out_specs=[pl.BlockSpec((1, 1, _N), lambda b: (b, 0, 0)),
                   pl.BlockSpec((1, 1, _N), lambda b: (b, 0, 0))],
        out_shape=[jax.ShapeDtypeStruct((_B, 1, _N), jnp.float32),
                   jax.ShapeDtypeStruct((_B, 1, _N), jnp.int32)],
    )(scores_t)
    best = best.reshape(_B, _N)
    cls = cls.reshape(_B, _N)
    boxes_t = jnp.transpose(boxes, (2, 0, 1))

    outs = pl.pallas_call(
        _nms_body,
        out_shape=[jax.ShapeDtypeStruct((_B, _D), jnp.int32),
                   jax.ShapeDtypeStruct((_B, _D), jnp.float32),
                   jax.ShapeDtypeStruct((_B, _D), jnp.float32),
                   jax.ShapeDtypeStruct((_B, _D), jnp.float32),
                   jax.ShapeDtypeStruct((_B, _D), jnp.float32),
                   jax.ShapeDtypeStruct((_B, _D), jnp.float32),
                   jax.ShapeDtypeStruct((_B, _D), jnp.int32),
                   jax.ShapeDtypeStruct((_B, 1), jnp.int32)],
    )(best, cls, boxes_t)
    oidx, osc, ox1, oy1, ox2, oy2, ocls, cnt = outs
    boxes_out = jnp.stack([ox1, oy1, ox2, oy2], axis=-1)
    return oidx, osc, boxes_out, ocls, cnt.reshape(_B)
